# initial kernel scaffold (unmeasured)
import functools

import jax
import jax.numpy as jnp
from jax import lax
from jax.experimental import pallas as pl
from jax.experimental.pallas import tpu as pltpu

B, S, D = 1, 1024, 2048
DC = 128
H, Dh, Dr = 16, 128, 32
SCALE = (Dh + Dr) ** -0.5
F32 = jnp.float32


def _kv_exchange(x2, Wdkv, Wuk, Wuv):

    def body(x_ref, wdkv_ref, wuk_ref, wuv_ref, k_ref, v_ref,
             c_ref, cr_ref, wukr_ref, wuvr_ref, send_sems, recv_sems):
        my_x = lax.axis_index("x")
        my_y = lax.axis_index("y")
        my_z = lax.axis_index("z")
        partner = (my_x, 1 - my_y, my_z)

        barrier_sem = pltpu.get_barrier_semaphore()
        pl.semaphore_signal(barrier_sem, inc=1, device_id=partner,
                            device_id_type=pl.DeviceIdType.MESH)
        pl.semaphore_wait(barrier_sem, 1)

        c_ref[...] = jnp.dot(x_ref[...], wdkv_ref[...],
                             preferred_element_type=F32)

        rdmas = []
        for i, (src, dst) in enumerate(
            [(c_ref, cr_ref), (wuk_ref, wukr_ref), (wuv_ref, wuvr_ref)]
        ):
            rdma = pltpu.make_async_remote_copy(
                src_ref=src, dst_ref=dst,
                send_sem=send_sems.at[i], recv_sem=recv_sems.at[i],
                device_id=partner, device_id_type=pl.DeviceIdType.MESH,
            )
            rdma.start()
            rdmas.append(rdma)

        k_ref[...] = jnp.dot(c_ref[...], wuk_ref[...],
                             preferred_element_type=F32)
        v_ref[...] = jnp.dot(c_ref[...], wuv_ref[...],
                             preferred_element_type=F32)

        for rdma in rdmas:
            rdma.wait()

        k_ref[...] += jnp.dot(cr_ref[...], wukr_ref[...],
                              preferred_element_type=F32)
        v_ref[...] += jnp.dot(cr_ref[...], wuvr_ref[...],
                              preferred_element_type=F32)

    return pl.pallas_call(
        body,
        out_shape=[
            jax.ShapeDtypeStruct((S, D), F32),
            jax.ShapeDtypeStruct((S, D), F32),
        ],
        in_specs=[pl.BlockSpec(memory_space=pltpu.VMEM)] * 4,
        out_specs=[pl.BlockSpec(memory_space=pltpu.VMEM)] * 2,
        scratch_shapes=[
            pltpu.VMEM((S, DC), F32),
            pltpu.VMEM((S, DC), F32),
            pltpu.VMEM((DC, D), F32),
            pltpu.VMEM((DC, D), F32),
            pltpu.SemaphoreType.DMA((3,)),
            pltpu.SemaphoreType.DMA((3,)),
        ],
        compiler_params=pltpu.CompilerParams(collective_id=0),
    )(x2, Wdkv, Wuk, Wuv)


def _qproj(x2, Wq, Wqr, Wkr):
    def body(x_ref, wq_ref, wqr_ref, wkr_ref, q_ref, qr_ref, kr_ref):
        q_ref[...] = jnp.dot(x_ref[...], wq_ref[...],
                             preferred_element_type=F32)
        qr_ref[...] = jnp.dot(x_ref[...], wqr_ref[...],
                              preferred_element_type=F32)
        kr_ref[...] = jnp.dot(x_ref[...], wkr_ref[...],
                              preferred_element_type=F32)

    return pl.pallas_call(
        body,
        out_shape=[
            jax.ShapeDtypeStruct((S, H * Dh), F32),
            jax.ShapeDtypeStruct((S, H * Dr), F32),
            jax.ShapeDtypeStruct((S, Dr), F32),
        ],
        in_specs=[pl.BlockSpec(memory_space=pltpu.VMEM)] * 4,
        out_specs=[pl.BlockSpec(memory_space=pltpu.VMEM)] * 3,
    )(x2, Wq, Wqr, Wkr)


def _attn(Q, K, V, Qr, Kr):

    def body(q_ref, k_ref, v_ref, qr_ref, kr_ref, o_ref):
        s = lax.dot_general(q_ref[...], k_ref[...],
                            (((1,), (1,)), ((), ())),
                            preferred_element_type=F32)
        s += lax.dot_general(qr_ref[...], kr_ref[...],
                             (((1,), (1,)), ((), ())),
                             preferred_element_type=F32)
        s *= SCALE
        m = jnp.max(s, axis=-1, keepdims=True)
        p = jnp.exp(s - m)
        p = p / jnp.sum(p, axis=-1, keepdims=True)
        o_ref[...] = jnp.dot(p, v_ref[...], preferred_element_type=F32)

    return pl.pallas_call(
        body,
        grid=(H,),
        in_specs=[
            pl.BlockSpec((S, Dh), lambda h: (0, h)),
            pl.BlockSpec((S, Dh), lambda h: (0, h)),
            pl.BlockSpec((S, Dh), lambda h: (0, h)),
            pl.BlockSpec((S, Dr), lambda h: (0, h)),
            pl.BlockSpec((S, Dr), lambda h: (0, 0)),
        ],
        out_specs=pl.BlockSpec((S, Dh), lambda h: (0, h)),
        out_shape=jax.ShapeDtypeStruct((S, H * Dh), F32),
    )(Q, K, V, Qr, Kr)


def _oproj(O, Wo):
    def body(o_ref, wo_ref, out_ref):
        out_ref[...] = jnp.dot(o_ref[...], wo_ref[...],
                               preferred_element_type=F32)

    return pl.pallas_call(
        body,
        out_shape=jax.ShapeDtypeStruct((S, D), F32),
        in_specs=[pl.BlockSpec(memory_space=pltpu.VMEM)] * 2,
        out_specs=pl.BlockSpec(memory_space=pltpu.VMEM),
    )(O, Wo)


def kernel(x, Wdkv, Wuk, Wuv, Wq, Wqr, Wkr, Wo):
    x2 = x.reshape(S, D)
    K, V = _kv_exchange(x2, Wdkv, Wuk, Wuv)
    Q, Qr, Kr = _qproj(x2, Wq, Wqr, Wkr)
    O = _attn(Q, K, V, Qr, Kr)
    out = _oproj(O, Wo)
    return out.reshape(B, S, D)


# baseline (device time: 191946 ns/iter reference)
import functools

import jax
import jax.numpy as jnp
from jax import lax
from jax.experimental import pallas as pl
from jax.experimental.pallas import tpu as pltpu

B, S, D = 1, 1024, 2048
DC = 128
H, Dh, Dr = 16, 128, 32
SCALE = (Dh + Dr) ** -0.5
F32 = jnp.float32


def _kv_exchange(x2, Wdkv, Wuk, Wuv):

    def body(x_ref, wdkv_ref, wuk_ref, wuv_ref, k_ref, v_ref,
             c_ref, cr_ref, wukr_ref, wuvr_ref, send_sems, recv_sems):
        my_x = lax.axis_index("x")
        my_y = lax.axis_index("y")
        my_z = lax.axis_index("z")
        partner = (my_x, 1 - my_y, my_z)

        barrier_sem = pltpu.get_barrier_semaphore()
        pl.semaphore_signal(barrier_sem, inc=1, device_id=partner,
                            device_id_type=pl.DeviceIdType.MESH)
        pl.semaphore_wait(barrier_sem, 1)

        c_ref[...] = jnp.dot(x_ref[...], wdkv_ref[...],
                             preferred_element_type=F32)

        rdmas = []
        for i, (src, dst) in enumerate(
            [(c_ref, cr_ref), (wuk_ref, wukr_ref), (wuv_ref, wuvr_ref)]
        ):
            rdma = pltpu.make_async_remote_copy(
                src_ref=src, dst_ref=dst,
                send_sem=send_sems.at[i], recv_sem=recv_sems.at[i],
                device_id=partner, device_id_type=pl.DeviceIdType.MESH,
            )
            rdma.start()
            rdmas.append(rdma)

        k_ref[...] = jnp.dot(c_ref[...], wuk_ref[...],
                             preferred_element_type=F32)
        v_ref[...] = jnp.dot(c_ref[...], wuv_ref[...],
                             preferred_element_type=F32)

        for rdma in rdmas:
            rdma.wait()

        k_ref[...] += jnp.dot(cr_ref[...], wukr_ref[...],
                              preferred_element_type=F32)
        v_ref[...] += jnp.dot(cr_ref[...], wuvr_ref[...],
                              preferred_element_type=F32)

    return pl.pallas_call(
        body,
        out_shape=[
            jax.ShapeDtypeStruct((S, D), F32),
            jax.ShapeDtypeStruct((S, D), F32),
        ],
        in_specs=[pl.BlockSpec(memory_space=pltpu.VMEM)] * 4,
        out_specs=[pl.BlockSpec(memory_space=pltpu.VMEM)] * 2,
        scratch_shapes=[
            pltpu.VMEM((S, DC), F32),
            pltpu.VMEM((S, DC), F32),
            pltpu.VMEM((DC, D), F32),
            pltpu.VMEM((DC, D), F32),
            pltpu.SemaphoreType.DMA((3,)),
            pltpu.SemaphoreType.DMA((3,)),
        ],
        compiler_params=pltpu.CompilerParams(collective_id=0),
    )(x2, Wdkv, Wuk, Wuv)


def _qproj(x2, Wq, Wkr):
    def body(x_ref, wq_ref, wkr_ref, q_ref, kr_ref):
        q_ref[...] = jnp.dot(x_ref[...], wq_ref[...],
                             preferred_element_type=F32)
        kr_ref[...] = jnp.dot(x_ref[...], wkr_ref[...],
                              preferred_element_type=F32)

    return pl.pallas_call(
        body,
        out_shape=[
            jax.ShapeDtypeStruct((S, H * Dh), F32),
            jax.ShapeDtypeStruct((S, Dr), F32),
        ],
        in_specs=[pl.BlockSpec(memory_space=pltpu.VMEM)] * 3,
        out_specs=[pl.BlockSpec(memory_space=pltpu.VMEM)] * 2,
    )(x2, Wq, Wkr)


def _attn(Q, K, V, x2, WqrT, Kr):

    def body(q_ref, k_ref, v_ref, x_ref, wqrt_ref, kr_ref, o_ref):
        qr = lax.dot_general(x_ref[...], wqrt_ref[...],
                             (((1,), (1,)), ((), ())),
                             preferred_element_type=F32)
        s = lax.dot_general(q_ref[...], k_ref[...],
                            (((1,), (1,)), ((), ())),
                            preferred_element_type=F32)
        s += lax.dot_general(qr, kr_ref[...],
                             (((1,), (1,)), ((), ())),
                             preferred_element_type=F32)
        s *= SCALE
        m = jnp.max(s, axis=-1, keepdims=True)
        p = jnp.exp(s - m)
        p = p / jnp.sum(p, axis=-1, keepdims=True)
        o_ref[...] = jnp.dot(p, v_ref[...], preferred_element_type=F32)

    return pl.pallas_call(
        body,
        grid=(H,),
        in_specs=[
            pl.BlockSpec((S, Dh), lambda h: (0, h)),
            pl.BlockSpec((S, Dh), lambda h: (0, h)),
            pl.BlockSpec((S, Dh), lambda h: (0, h)),
            pl.BlockSpec((S, D), lambda h: (0, 0)),
            pl.BlockSpec((Dr, D), lambda h: (h, 0)),
            pl.BlockSpec((S, Dr), lambda h: (0, 0)),
        ],
        out_specs=pl.BlockSpec((S, Dh), lambda h: (0, h)),
        out_shape=jax.ShapeDtypeStruct((S, H * Dh), F32),
    )(Q, K, V, x2, WqrT, Kr)


def _oproj(O, Wo):
    def body(o_ref, wo_ref, out_ref):
        out_ref[...] = jnp.dot(o_ref[...], wo_ref[...],
                               preferred_element_type=F32)

    return pl.pallas_call(
        body,
        out_shape=jax.ShapeDtypeStruct((S, D), F32),
        in_specs=[pl.BlockSpec(memory_space=pltpu.VMEM)] * 2,
        out_specs=pl.BlockSpec(memory_space=pltpu.VMEM),
    )(O, Wo)


def kernel(x, Wdkv, Wuk, Wuv, Wq, Wqr, Wkr, Wo):
    x2 = x.reshape(S, D)
    K, V = _kv_exchange(x2, Wdkv, Wuk, Wuv)
    Q, Kr = _qproj(x2, Wq, Wkr)
    O = _attn(Q, K, V, x2, Wqr.T, Kr)
    out = _oproj(O, Wo)
    return out.reshape(B, S, D)


# device time: 188470 ns/iter; 1.0184x vs baseline; 1.0184x over previous
import jax
import jax.numpy as jnp
from jax import lax
from jax.experimental import pallas as pl
from jax.experimental.pallas import tpu as pltpu

B, S, D = 1, 1024, 2048
DC = 128
H, Dh, Dr = 16, 128, 32
SCALE = (Dh + Dr) ** -0.5
F32 = jnp.float32


def _kv_exchange(x2, Wdkv, Wuk, Wuv, Wkr):

    def body(x_ref, wdkv_ref, wuk_ref, wuv_ref, wkr_ref,
             k_ref, v_ref, kr_ref,
             c_ref, cr_ref, wukr_ref, wuvr_ref, send_sems, recv_sems):
        my_x = lax.axis_index("x")
        my_y = lax.axis_index("y")
        my_z = lax.axis_index("z")
        partner = (my_x, 1 - my_y, my_z)

        barrier_sem = pltpu.get_barrier_semaphore()
        pl.semaphore_signal(barrier_sem, inc=1, device_id=partner,
                            device_id_type=pl.DeviceIdType.MESH)
        pl.semaphore_wait(barrier_sem, 1)

        c_ref[...] = jnp.dot(x_ref[...], wdkv_ref[...],
                             preferred_element_type=F32)

        rdmas = []
        for i, (src, dst) in enumerate(
            [(c_ref, cr_ref), (wuk_ref, wukr_ref), (wuv_ref, wuvr_ref)]
        ):
            rdma = pltpu.make_async_remote_copy(
                src_ref=src, dst_ref=dst,
                send_sem=send_sems.at[i], recv_sem=recv_sems.at[i],
                device_id=partner, device_id_type=pl.DeviceIdType.MESH,
            )
            rdma.start()
            rdmas.append(rdma)

        kr_ref[...] = jnp.dot(x_ref[...], wkr_ref[...],
                              preferred_element_type=F32)
        k_ref[...] = jnp.dot(c_ref[...], wuk_ref[...],
                             preferred_element_type=F32)
        v_ref[...] = jnp.dot(c_ref[...], wuv_ref[...],
                             preferred_element_type=F32)

        for rdma in rdmas:
            rdma.wait()

        k_ref[...] += jnp.dot(cr_ref[...], wukr_ref[...],
                              preferred_element_type=F32)
        v_ref[...] += jnp.dot(cr_ref[...], wuvr_ref[...],
                              preferred_element_type=F32)

    return pl.pallas_call(
        body,
        out_shape=[
            jax.ShapeDtypeStruct((S, D), F32),
            jax.ShapeDtypeStruct((S, D), F32),
            jax.ShapeDtypeStruct((S, Dr), F32),
        ],
        in_specs=[pl.BlockSpec(memory_space=pltpu.VMEM)] * 5,
        out_specs=[pl.BlockSpec(memory_space=pltpu.VMEM)] * 3,
        scratch_shapes=[
            pltpu.VMEM((S, DC), F32),
            pltpu.VMEM((S, DC), F32),
            pltpu.VMEM((DC, D), F32),
            pltpu.VMEM((DC, D), F32),
            pltpu.SemaphoreType.DMA((3,)),
            pltpu.SemaphoreType.DMA((3,)),
        ],
        compiler_params=pltpu.CompilerParams(collective_id=0),
    )(x2, Wdkv, Wuk, Wuv, Wkr)


def _attn_out(K, V, x2, Wq, WqrT, Kr, Wo):

    def body(k_ref, v_ref, x_ref, wq_ref, wqrt_ref, kr_ref, wo_ref,
             out_ref):
        h = pl.program_id(0)
        q = jnp.dot(x_ref[...], wq_ref[...], preferred_element_type=F32)
        qr = lax.dot_general(x_ref[...], wqrt_ref[...],
                             (((1,), (1,)), ((), ())),
                             preferred_element_type=F32)
        s = lax.dot_general(q, k_ref[...],
                            (((1,), (1,)), ((), ())),
                            preferred_element_type=F32)
        s += lax.dot_general(qr, kr_ref[...],
                             (((1,), (1,)), ((), ())),
                             preferred_element_type=F32)
        s *= SCALE
        m = jnp.max(s, axis=-1, keepdims=True)
        p = jnp.exp(s - m)
        p = p / jnp.sum(p, axis=-1, keepdims=True)
        o_h = jnp.dot(p, v_ref[...], preferred_element_type=F32)
        contrib = jnp.dot(o_h, wo_ref[...], preferred_element_type=F32)

        @pl.when(h == 0)
        def _():
            out_ref[...] = contrib

        @pl.when(h > 0)
        def _():
            out_ref[...] += contrib

    return pl.pallas_call(
        body,
        grid=(H,),
        in_specs=[
            pl.BlockSpec((S, Dh), lambda h: (0, h)),
            pl.BlockSpec((S, Dh), lambda h: (0, h)),
            pl.BlockSpec((S, D), lambda h: (0, 0)),
            pl.BlockSpec((D, Dh), lambda h: (0, h)),
            pl.BlockSpec((Dr, D), lambda h: (h, 0)),
            pl.BlockSpec((S, Dr), lambda h: (0, 0)),
            pl.BlockSpec((Dh, D), lambda h: (h, 0)),
        ],
        out_specs=pl.BlockSpec((S, D), lambda h: (0, 0)),
        out_shape=jax.ShapeDtypeStruct((S, D), F32),
    )(K, V, x2, Wq, WqrT, Kr, Wo)


def kernel(x, Wdkv, Wuk, Wuv, Wq, Wqr, Wkr, Wo):
    x2 = x.reshape(S, D)
    K, V, Kr = _kv_exchange(x2, Wdkv, Wuk, Wuv, Wkr)
    out = _attn_out(K, V, x2, Wq, Wqr.T, Kr, Wo)
    return out.reshape(B, S, D)


# device time: 129789 ns/iter; 1.4789x vs baseline; 1.4521x over previous
import jax
import jax.numpy as jnp
from jax import lax
from jax.experimental import pallas as pl
from jax.experimental.pallas import tpu as pltpu

B, S, D = 1, 1024, 2048
DC = 128
H, Dh, Dr = 16, 128, 32
HP = H // 2
SCALE = (Dh + Dr) ** -0.5
F32 = jnp.float32
BF16 = jnp.bfloat16


def _kv_exchange(x2, Wdkv, Wuk, Wuv, Wkr):

    def body(x_ref, wdkv_ref, wuk_ref, wuv_ref, wkr_ref,
             k_ref, v_ref, kr_ref,
             c_ref, cr_ref, wukb_ref, wuvb_ref, wukr_ref, wuvr_ref,
             send_sems, recv_sems):
        my_x = lax.axis_index("x")
        my_y = lax.axis_index("y")
        my_z = lax.axis_index("z")
        partner = (my_x, 1 - my_y, my_z)

        barrier_sem = pltpu.get_barrier_semaphore()
        pl.semaphore_signal(barrier_sem, inc=1, device_id=partner,
                            device_id_type=pl.DeviceIdType.MESH)
        pl.semaphore_wait(barrier_sem, 1)

        c32 = jnp.dot(x_ref[...], wdkv_ref[...], preferred_element_type=F32)
        c_ref[...] = c32.astype(BF16)
        wukb_ref[...] = wuk_ref[...].astype(BF16)
        wuvb_ref[...] = wuv_ref[...].astype(BF16)

        rdmas = []
        for i, (src, dst) in enumerate(
            [(c_ref, cr_ref), (wukb_ref, wukr_ref), (wuvb_ref, wuvr_ref)]
        ):
            rdma = pltpu.make_async_remote_copy(
                src_ref=src, dst_ref=dst,
                send_sem=send_sems.at[i], recv_sem=recv_sems.at[i],
                device_id=partner, device_id_type=pl.DeviceIdType.MESH,
            )
            rdma.start()
            rdmas.append(rdma)

        kr_ref[...] = jnp.dot(x_ref[...], wkr_ref[...],
                              preferred_element_type=F32)
        k32 = jnp.dot(c32, wuk_ref[...], preferred_element_type=F32)
        v32 = jnp.dot(c32, wuv_ref[...], preferred_element_type=F32)

        for rdma in rdmas:
            rdma.wait()

        k32 += jnp.dot(cr_ref[...], wukr_ref[...], preferred_element_type=F32)
        v32 += jnp.dot(cr_ref[...], wuvr_ref[...], preferred_element_type=F32)
        k_ref[...] = k32.astype(BF16)
        v_ref[...] = v32.astype(BF16)

    return pl.pallas_call(
        body,
        out_shape=[
            jax.ShapeDtypeStruct((S, D), BF16),
            jax.ShapeDtypeStruct((S, D), BF16),
            jax.ShapeDtypeStruct((S, Dr), F32),
        ],
        in_specs=[pl.BlockSpec(memory_space=pltpu.VMEM)] * 5,
        out_specs=[pl.BlockSpec(memory_space=pltpu.VMEM)] * 3,
        scratch_shapes=[
            pltpu.VMEM((S, DC), BF16),
            pltpu.VMEM((S, DC), BF16),
            pltpu.VMEM((DC, D), BF16),
            pltpu.VMEM((DC, D), BF16),
            pltpu.VMEM((DC, D), BF16),
            pltpu.VMEM((DC, D), BF16),
            pltpu.SemaphoreType.DMA((3,)),
            pltpu.SemaphoreType.DMA((3,)),
        ],
        compiler_params=pltpu.CompilerParams(collective_id=0),
    )(x2, Wdkv, Wuk, Wuv, Wkr)


def _attn_out(K, V, x2, Wq, WqrT, Kr, Wo):

    def body(k_ref, v_ref, x_ref, wq_ref, wqrt_ref, kr_ref, wo_ref,
             out_ref):
        hp = pl.program_id(0)
        q2 = jnp.dot(x_ref[...], wq_ref[...], preferred_element_type=F32)
        o_halves = []
        for j in (0, 1):
            q = q2[:, j * Dh:(j + 1) * Dh]
            k = k_ref[:, j * Dh:(j + 1) * Dh]
            v = v_ref[:, j * Dh:(j + 1) * Dh]
            wqrt = wqrt_ref[j * Dr:(j + 1) * Dr, :]
            qr = lax.dot_general(x_ref[...], wqrt,
                                 (((1,), (1,)), ((), ())),
                                 preferred_element_type=F32)
            s = lax.dot_general(q, k, (((1,), (1,)), ((), ())),
                                preferred_element_type=F32)
            s += lax.dot_general(qr, kr_ref[...],
                                 (((1,), (1,)), ((), ())),
                                 preferred_element_type=F32)
            s *= SCALE
            m = jnp.max(s, axis=-1, keepdims=True)
            p = jnp.exp(s - m)
            r = 1.0 / jnp.sum(p, axis=-1, keepdims=True)
            o = jnp.dot(p, v, preferred_element_type=F32) * r
            o_halves.append(o)
        o2 = jnp.concatenate(o_halves, axis=1)
        contrib = jnp.dot(o2, wo_ref[...], preferred_element_type=F32)

        @pl.when(hp == 0)
        def _():
            out_ref[...] = contrib

        @pl.when(hp > 0)
        def _():
            out_ref[...] += contrib

    return pl.pallas_call(
        body,
        grid=(HP,),
        in_specs=[
            pl.BlockSpec((S, 2 * Dh), lambda h: (0, h)),
            pl.BlockSpec((S, 2 * Dh), lambda h: (0, h)),
            pl.BlockSpec((S, D), lambda h: (0, 0)),
            pl.BlockSpec((D, 2 * Dh), lambda h: (0, h)),
            pl.BlockSpec((2 * Dr, D), lambda h: (h, 0)),
            pl.BlockSpec((S, Dr), lambda h: (0, 0)),
            pl.BlockSpec((2 * Dh, D), lambda h: (h, 0)),
        ],
        out_specs=pl.BlockSpec((S, D), lambda h: (0, 0)),
        out_shape=jax.ShapeDtypeStruct((S, D), F32),
    )(K, V, x2, Wq, WqrT, Kr, Wo)


def kernel(x, Wdkv, Wuk, Wuv, Wq, Wqr, Wkr, Wo):
    x2 = x.reshape(S, D)
    K, V, Kr = _kv_exchange(x2, Wdkv, Wuk, Wuv, Wkr)
    out = _attn_out(K, V, x2, Wq, Wqr.T, Kr, Wo)
    return out.reshape(B, S, D)


# device time: 114157 ns/iter; 1.6814x vs baseline; 1.1369x over previous
import jax
import jax.numpy as jnp
from jax import lax
from jax.experimental import pallas as pl
from jax.experimental.pallas import tpu as pltpu

B, S, D = 1, 1024, 2048
DC = 128
H, Dh, Dr = 16, 128, 32
HG = 4
NG = H // HG
SCALE = (Dh + Dr) ** -0.5
F32 = jnp.float32
BF16 = jnp.bfloat16


def _kv_exchange(x2, Wdkv, Wuk, Wuv, Wkr):

    def body(x_ref, wdkv_ref, wuk_ref, wuv_ref, wkr_ref,
             k_ref, v_ref, kr_ref,
             c_ref, cr_ref, wukb_ref, wuvb_ref, wukr_ref, wuvr_ref,
             send_sems, recv_sems):
        my_x = lax.axis_index("x")
        my_y = lax.axis_index("y")
        my_z = lax.axis_index("z")
        partner = (my_x, 1 - my_y, my_z)

        barrier_sem = pltpu.get_barrier_semaphore()
        pl.semaphore_signal(barrier_sem, inc=1, device_id=partner,
                            device_id_type=pl.DeviceIdType.MESH)
        pl.semaphore_wait(barrier_sem, 1)

        c32 = jnp.dot(x_ref[...], wdkv_ref[...], preferred_element_type=F32)
        c_ref[...] = c32.astype(BF16)
        wukb_ref[...] = wuk_ref[...].astype(BF16)
        wuvb_ref[...] = wuv_ref[...].astype(BF16)

        rdmas = []
        for i, (src, dst) in enumerate(
            [(c_ref, cr_ref), (wukb_ref, wukr_ref), (wuvb_ref, wuvr_ref)]
        ):
            rdma = pltpu.make_async_remote_copy(
                src_ref=src, dst_ref=dst,
                send_sem=send_sems.at[i], recv_sem=recv_sems.at[i],
                device_id=partner, device_id_type=pl.DeviceIdType.MESH,
            )
            rdma.start()
            rdmas.append(rdma)

        kr_ref[...] = jnp.dot(x_ref[...], wkr_ref[...],
                              preferred_element_type=F32)
        k32 = jnp.dot(c32, wuk_ref[...], preferred_element_type=F32)
        v32 = jnp.dot(c32, wuv_ref[...], preferred_element_type=F32)

        for rdma in rdmas:
            rdma.wait()

        k32 += jnp.dot(cr_ref[...], wukr_ref[...], preferred_element_type=F32)
        v32 += jnp.dot(cr_ref[...], wuvr_ref[...], preferred_element_type=F32)
        k_ref[...] = k32.astype(BF16)
        v_ref[...] = v32.astype(BF16)

    return pl.pallas_call(
        body,
        out_shape=[
            jax.ShapeDtypeStruct((S, D), BF16),
            jax.ShapeDtypeStruct((S, D), BF16),
            jax.ShapeDtypeStruct((S, Dr), F32),
        ],
        in_specs=[pl.BlockSpec(memory_space=pltpu.VMEM)] * 5,
        out_specs=[pl.BlockSpec(memory_space=pltpu.VMEM)] * 3,
        scratch_shapes=[
            pltpu.VMEM((S, DC), BF16),
            pltpu.VMEM((S, DC), BF16),
            pltpu.VMEM((DC, D), BF16),
            pltpu.VMEM((DC, D), BF16),
            pltpu.VMEM((DC, D), BF16),
            pltpu.VMEM((DC, D), BF16),
            pltpu.SemaphoreType.DMA((3,)),
            pltpu.SemaphoreType.DMA((3,)),
        ],
        compiler_params=pltpu.CompilerParams(collective_id=0),
    )(x2, Wdkv, Wuk, Wuv, Wkr)


def _attn_out(K, V, x2, Wq, WqrT, Kr, Wo):

    def body(k_ref, v_ref, x_ref, wq_ref, wqrt_ref, kr_ref, wo_ref,
             out_ref):
        hp = pl.program_id(0)
        qg = jnp.dot(x_ref[...], wq_ref[...], preferred_element_type=F32)
        qrg = lax.dot_general(x_ref[...], wqrt_ref[...],
                              (((1,), (1,)), ((), ())),
                              preferred_element_type=F32)
        kr = kr_ref[...]
        o_parts = []
        for j in range(HG):
            q = qg[:, j * Dh:(j + 1) * Dh]
            qr = qrg[:, j * Dr:(j + 1) * Dr]
            k = k_ref[:, j * Dh:(j + 1) * Dh]
            v = v_ref[:, j * Dh:(j + 1) * Dh]
            qa = jnp.concatenate([q, qr], axis=1)
            ka = jnp.concatenate([k.astype(F32), kr], axis=1)
            s = lax.dot_general(qa, ka, (((1,), (1,)), ((), ())),
                                preferred_element_type=F32)
            s *= SCALE
            m = jnp.max(s, axis=-1, keepdims=True)
            p = jnp.exp(s - m)
            r = 1.0 / jnp.sum(p, axis=-1, keepdims=True)
            o = jnp.dot(p, v, preferred_element_type=F32) * r
            o_parts.append(o)
        og = jnp.concatenate(o_parts, axis=1)
        contrib = jnp.dot(og, wo_ref[...], preferred_element_type=F32)

        @pl.when(hp == 0)
        def _():
            out_ref[...] = contrib

        @pl.when(hp > 0)
        def _():
            out_ref[...] += contrib

    return pl.pallas_call(
        body,
        grid=(NG,),
        in_specs=[
            pl.BlockSpec((S, HG * Dh), lambda h: (0, h)),
            pl.BlockSpec((S, HG * Dh), lambda h: (0, h)),
            pl.BlockSpec((S, D), lambda h: (0, 0)),
            pl.BlockSpec((D, HG * Dh), lambda h: (0, h)),
            pl.BlockSpec((HG * Dr, D), lambda h: (h, 0)),
            pl.BlockSpec((S, Dr), lambda h: (0, 0)),
            pl.BlockSpec((HG * Dh, D), lambda h: (h, 0)),
        ],
        out_specs=pl.BlockSpec((S, D), lambda h: (0, 0)),
        out_shape=jax.ShapeDtypeStruct((S, D), F32),
        compiler_params=pltpu.CompilerParams(
            vmem_limit_bytes=128 * 1024 * 1024,
        ),
    )(K, V, x2, Wq, WqrT, Kr, Wo)


def kernel(x, Wdkv, Wuk, Wuv, Wq, Wqr, Wkr, Wo):
    x2 = x.reshape(S, D)
    K, V, Kr = _kv_exchange(x2, Wdkv, Wuk, Wuv, Wkr)
    out = _attn_out(K, V, x2, Wq, Wqr.T, Kr, Wo)
    return out.reshape(B, S, D)
